# baseline (speedup 1.0000x reference)
import jax
import jax.numpy as jnp
from jax import lax
from jax.experimental import pallas as pl
from jax.experimental.pallas import tpu as pltpu
from jax.experimental.pallas import tpu_sc as plsc

B, C, CP, CV, H, W, NK = 8, 192, 64, 192, 32, 32, 8
N = H * W
K5 = NK * 5
W48 = 48
NTILES = 32
CH = 16
T2 = 2
NH = 2
B2 = B // NH
ROWS2 = B2 * N
RPT = ROWS2 // NTILES
NCH = RPT // CH
TPB = N // RPT
BM = 256


def _k1_body(q_ref, k_ref, qp_ref, kp_ref, a_ref):
    dn = (((0,), (0,)), ((), ()))
    a = lax.dot_general(q_ref[0], k_ref[0], dn,
                        preferred_element_type=jnp.float32)
    p = lax.dot_general(qp_ref[0], kp_ref[0], dn,
                        preferred_element_type=jnp.float32)
    ai = lax.bitcast_convert_type(a, jnp.int32)
    enc = jnp.where(p > 0, ai | 1, ai & -2)
    a_ref[0] = lax.bitcast_convert_type(enc, jnp.float32)


def _k1(q_r, k_r, qp_r, kp_r, b0):
    return pl.pallas_call(
        _k1_body,
        grid=(B2, N // BM),
        in_specs=[
            pl.BlockSpec((1, C, BM), lambda b, i: (b, 0, i)),
            pl.BlockSpec((1, C, N), lambda b, i: (b, 0, 0)),
            pl.BlockSpec((1, CP, BM), lambda b, i: (b, 0, i)),
            pl.BlockSpec((1, CP, N), lambda b, i: (b, 0, 0)),
        ],
        out_specs=pl.BlockSpec((1, BM, N), lambda b, i: (b, i, 0)),
        out_shape=jax.ShapeDtypeStruct((B2, N, N), jnp.float32),
        compiler_params=pltpu.CompilerParams(
            dimension_semantics=("parallel", "parallel")),
    )(q_r, k_r, qp_r, kp_r)


def _k2_body(b0, prev_ref, a_hbm, idx_out, s_out,
             idx_vmem, a_buf0, a_buf1, idxbuf, sbuf, sem0, sem1):
    wid = lax.axis_index("c") * 16 + lax.axis_index("s")
    tile_base = wid * RPT
    batch = b0 + wid // TPB
    n0 = (wid % TPB) * RPT

    pltpu.sync_copy(prev_ref.at[pl.ds(batch * N * NK, N * NK)], idx_vmem)

    iota16 = lax.broadcasted_iota(jnp.int32, (16,), 0)

    def chunk_src(ci):
        return a_hbm.at[pl.ds(tile_base + ci * CH, CH)]

    def process(ci, a_chunk):
        n_vec = n0 + ci * CH + iota16
        y = lax.shift_right_logical(n_vec, 5)
        x = lax.bitwise_and(n_vec, 31)
        srcs = [
            n_vec,
            jnp.maximum(y - 1, 0) * 32 + x,
            jnp.minimum(y + 1, 31) * 32 + x,
            y * 32 + jnp.maximum(x - 1, 0),
            y * 32 + jnp.minimum(x + 1, 31),
        ]
        col0 = ci * CH
        for c in range(K5):
            g, t = c // 8, c % 8
            p_val = plsc.load_gather(idx_vmem, [srcs[g] * NK + t])
            py = lax.shift_right_logical(p_val, 5)
            px = lax.bitwise_and(p_val, 31)
            if g == 0:
                oidx = p_val
            elif g == 1:
                oidx = jnp.minimum(py + 1, 31) * 32 + px
            elif g == 2:
                oidx = jnp.maximum(py - 1, 0) * 32 + px
            elif g == 3:
                oidx = py * 32 + jnp.minimum(px + 1, 31)
            else:
                oidx = py * 32 + jnp.maximum(px - 1, 0)
            idxbuf[c, pl.ds(col0, 16)] = oidx
            sbuf[c, pl.ds(col0, 16)] = plsc.load_gather(a_chunk,
                                                        [iota16, oidx])

    pltpu.async_copy(chunk_src(0), a_buf0, sem0)
    pltpu.async_copy(chunk_src(1), a_buf1, sem1)

    def pair_body(i, carry):
        ci0 = 2 * i
        ci1 = 2 * i + 1
        pltpu.make_async_copy(chunk_src(ci0), a_buf0, sem0).wait()
        process(ci0, a_buf0)
        pltpu.async_copy(chunk_src(jnp.minimum(ci0 + 2, NCH - 1)), a_buf0,
                         sem0)
        pltpu.make_async_copy(chunk_src(ci1), a_buf1, sem1).wait()
        process(ci1, a_buf1)
        pltpu.async_copy(chunk_src(jnp.minimum(ci1 + 2, NCH - 1)), a_buf1,
                         sem1)
        return carry

    lax.fori_loop(0, NCH // 2, pair_body, 0)
    pltpu.make_async_copy(chunk_src(NCH - 1), a_buf0, sem0).wait()
    pltpu.make_async_copy(chunk_src(NCH - 1), a_buf1, sem1).wait()

    zero16 = jnp.zeros((16,), jnp.int32)
    zf16 = jnp.zeros((16,), jnp.float32)
    for c in range(K5, W48):
        for j in range(RPT // 16):
            idxbuf[c, pl.ds(j * 16, 16)] = zero16
            sbuf[c, pl.ds(j * 16, 16)] = zf16

    pltpu.sync_copy(idxbuf, idx_out.at[wid])
    pltpu.sync_copy(sbuf, s_out.at[wid])


def _k2(prev_flat, a_half, b0):
    mesh = plsc.VectorSubcoreMesh(core_axis_name="c", subcore_axis_name="s")
    return pl.kernel(
        lambda *refs: _k2_body(b0, *refs),
        out_type=[
            jax.ShapeDtypeStruct((NTILES, W48, RPT), jnp.int32),
            jax.ShapeDtypeStruct((NTILES, W48, RPT), jnp.float32),
        ],
        mesh=mesh,
        compiler_params=pltpu.CompilerParams(needs_layout_passes=False),
        scratch_types=[
            pltpu.VMEM((N * NK,), jnp.int32),
            pltpu.VMEM((CH, N), jnp.float32),
            pltpu.VMEM((CH, N), jnp.float32),
            pltpu.VMEM((W48, RPT), jnp.int32),
            pltpu.VMEM((W48, RPT), jnp.float32),
            pltpu.SemaphoreType.DMA,
            pltpu.SemaphoreType.DMA,
        ],
    )(prev_flat, a_half)


ROWS_ALL = NH * ROWS2


def _k3_body(s_ref, i_ref, w_ref, scat_ref, topk_ref):
    s_enc = lax.bitcast_convert_type(s_ref[...], jnp.int32)
    pmask = (s_enc & 1) == 1
    s = lax.bitcast_convert_type(s_enc & -2, jnp.float32)
    idx = i_ref[...]
    sub = lax.broadcasted_iota(jnp.int32, (T2, W48, RPT), 1)
    tmask = sub < K5

    smooth = jnp.float32(C ** 0.5)
    s_m = jnp.where(tmask, s / smooth, -jnp.inf)
    mx = jnp.max(s_m, axis=1, keepdims=True)
    e = jnp.exp(s_m - mx)
    denom = jnp.sum(e, axis=1, keepdims=True)
    attn = e / denom
    w = jnp.where(tmask & pmask, attn, 0.0)
    conf = jnp.sum(w, axis=1, keepdims=True)

    vals = jnp.where(tmask, attn, -1.0)
    accw = jnp.zeros((T2, W48, RPT), jnp.float32)
    minpos = jnp.full((T2, W48, RPT), W48, jnp.int32)
    rank = jnp.zeros((T2, W48, RPT), jnp.int32)
    for tp in range(K5):
        idx_tp = idx[:, tp:tp + 1, :]
        v_tp = vals[:, tp:tp + 1, :]
        eq = idx == idx_tp
        accw = accw + jnp.where(eq, w[:, tp:tp + 1, :], 0.0)
        minpos = jnp.minimum(minpos, jnp.where(eq, tp, W48))
        beats = (v_tp > vals) | ((v_tp == vals) & (tp < sub))
        rank = rank + beats.astype(jnp.int32)
    first = (minpos == sub) & tmask
    w_comb = jnp.where(first, accw, 0.0)
    scat_ref[...] = jnp.where(first, idx, -1)
    w_ref[...] = jnp.where(sub == (W48 - 1), conf, w_comb)

    rank = jnp.where(tmask, rank, W48)
    rows_k = [jnp.sum(jnp.where(rank == i, idx, 0), axis=1, keepdims=True)
              for i in range(NK)]
    tk = jnp.transpose(jnp.concatenate(rows_k, axis=1), (0, 2, 1))
    topk_ref[...] = tk.reshape(T2 * RPT, NK)


def _k3(s3, idx3, hh, topk_prev=None):
    off = hh * (NTILES // T2)
    body = _k3_body
    in_specs = [
        pl.BlockSpec((T2, W48, RPT), lambda i: (i, 0, 0)),
        pl.BlockSpec((T2, W48, RPT), lambda i: (i, 0, 0)),
    ]
    args = [s3, idx3]
    aliases = {}
    if topk_prev is not None:
        body = lambda s, i, tp, w, sc, tk: _k3_body(s, i, w, sc, tk)
        in_specs.append(pl.BlockSpec(memory_space=pl.ANY))
        args.append(topk_prev)
        aliases = {2: 2}
    return pl.pallas_call(
        body,
        grid=(NTILES // T2,),
        in_specs=in_specs,
        out_specs=[
            pl.BlockSpec((T2, W48, RPT), lambda i: (i, 0, 0)),
            pl.BlockSpec((T2, W48, RPT), lambda i: (i, 0, 0)),
            pl.BlockSpec((T2 * RPT, NK), lambda i: (i + off, 0)),
        ],
        out_shape=[
            jax.ShapeDtypeStruct((NTILES, W48, RPT), jnp.float32),
            jax.ShapeDtypeStruct((NTILES, W48, RPT), jnp.int32),
            jax.ShapeDtypeStruct((ROWS_ALL, NK), jnp.int32),
        ],
        input_output_aliases=aliases,
        compiler_params=pltpu.CompilerParams(
            dimension_semantics=("arbitrary",)),
    )(*args)


def _k4_body(w_hbm, scat_hbm, s_out, wbuf, scatbuf, s_chunk):
    wid = lax.axis_index("c") * 16 + lax.axis_index("s")
    tile_base = wid * RPT

    pltpu.sync_copy(w_hbm.at[wid], wbuf)
    pltpu.sync_copy(scat_hbm.at[wid], scatbuf)

    iota16 = lax.broadcasted_iota(jnp.int32, (16,), 0)
    zf16 = jnp.zeros((16,), jnp.float32)

    def zero_body(j, carry):
        for r in range(CH):
            s_chunk[r, pl.ds(j * 16, 16)] = zf16
        return carry

    lax.fori_loop(0, N // 16, zero_body, 0)

    def chunk_body(ci, carry):
        col0 = ci * CH
        for c in range(K5):
            w_vec = wbuf[c, pl.ds(col0, 16)]
            scat = scatbuf[c, pl.ds(col0, 16)]
            col = jnp.maximum(scat, 0)
            plsc.store_scatter(s_chunk, [iota16, col], w_vec, mask=scat >= 0)
        pltpu.sync_copy(s_chunk,
                        s_out.at[pl.ds(tile_base + ci * CH, CH)])
        for c in range(K5):
            scat = scatbuf[c, pl.ds(col0, 16)]
            col = jnp.maximum(scat, 0)
            plsc.store_scatter(s_chunk, [iota16, col], zf16, mask=scat >= 0)
        return carry

    lax.fori_loop(0, NCH, chunk_body, 0)


def _k4(w3, scat3):
    mesh = plsc.VectorSubcoreMesh(core_axis_name="c", subcore_axis_name="s")
    return pl.kernel(
        _k4_body,
        out_type=jax.ShapeDtypeStruct((ROWS2, N), jnp.float32),
        mesh=mesh,
        compiler_params=pltpu.CompilerParams(needs_layout_passes=False),
        scratch_types=[
            pltpu.VMEM((W48, RPT), jnp.float32),
            pltpu.VMEM((W48, RPT), jnp.int32),
            pltpu.VMEM((CH, N), jnp.float32),
        ],
    )(w3, scat3)


def _k5_body(v_ref, s_ref, o_ref):
    o_ref[0] = lax.dot_general(v_ref[0], s_ref[0], (((1,), (1,)), ((), ())),
                               preferred_element_type=jnp.float32)


def _k5(v_r, s_dense, b0, out_prev=None):
    body = _k5_body
    in_specs = [
        pl.BlockSpec((1, CV, N), lambda b, i: (b + b0, 0, 0)),
        pl.BlockSpec((1, BM, N), lambda b, i: (b, i, 0)),
    ]
    args = [v_r, s_dense]
    aliases = {}
    if out_prev is not None:
        body = lambda v, s, op, o: _k5_body(v, s, o)
        in_specs.append(pl.BlockSpec(memory_space=pl.ANY))
        args.append(out_prev)
        aliases = {2: 0}
    return pl.pallas_call(
        body,
        grid=(B2, N // BM),
        in_specs=in_specs,
        out_specs=pl.BlockSpec((1, CV, BM), lambda b, i: (b + b0, 0, i)),
        out_shape=jax.ShapeDtypeStruct((B, CV, N), jnp.float32),
        input_output_aliases=aliases,
        compiler_params=pltpu.CompilerParams(
            dimension_semantics=("arbitrary", "arbitrary")),
    )(*args)


@jax.jit
def kernel(q, k, q_prune, k_prune, v, prev_attn_top_k_idx):
    v_r = v.reshape(B, CV, N)
    prev_flat = prev_attn_top_k_idx.reshape(B * N * NK)

    out_full = None
    topk_full = None
    confs = []
    for hh in range(NH):
        b0 = hh * B2
        q_rh = q[b0:b0 + B2].reshape(B2, C, N)
        k_rh = k[b0:b0 + B2].reshape(B2, C, N)
        qp_rh = q_prune[b0:b0 + B2].reshape(B2, CP, N)
        kp_rh = k_prune[b0:b0 + B2].reshape(B2, CP, N)
        a_d = _k1(q_rh, k_rh, qp_rh, kp_rh, b0)
        idx3, s3 = _k2(prev_flat, a_d.reshape(ROWS2, N), b0)
        w3, scat3, topk_full = _k3(s3, idx3, hh, topk_full)
        s2 = _k4(w3, scat3)
        out_full = _k5(v_r, s2.reshape(B2, N, N), b0, out_full)
        confs.append(w3[:, W48 - 1, :])

    output = out_full.reshape(B, CV, H, W)
    this_attn_top_k_idx = topk_full.reshape(B, N, NK)
    conf = jnp.concatenate(confs, axis=0).reshape(B, 1, H, W)
    return output, this_attn_top_k_idx, conf


# speedup vs baseline: 1.1508x; 1.1508x over previous
import jax
import jax.numpy as jnp
from jax import lax
from jax.experimental import pallas as pl
from jax.experimental.pallas import tpu as pltpu
from jax.experimental.pallas import tpu_sc as plsc

B, C, CP, CV, H, W, NK = 8, 192, 64, 192, 32, 32, 8
N = H * W
K5 = NK * 5
W48 = 48
NTILES = 32
CH = 16
T2 = 2
NH = 2
B2 = B // NH
ROWS2 = B2 * N
RPT = ROWS2 // NTILES
NCH = RPT // CH
TPB = N // RPT
BM = 256


def _k1_body(q_ref, k_ref, qp_ref, kp_ref, a_ref):
    dn = (((0,), (0,)), ((), ()))
    a = lax.dot_general(q_ref[0], k_ref[0], dn,
                        preferred_element_type=jnp.float32)
    p = lax.dot_general(qp_ref[0], kp_ref[0], dn,
                        preferred_element_type=jnp.float32)
    ai = lax.bitcast_convert_type(a, jnp.int32)
    enc = jnp.where(p > 0, ai | 1, ai & -2)
    a_ref[0] = lax.bitcast_convert_type(enc, jnp.float32)


def _k1(q_r, k_r, qp_r, kp_r, b0):
    return pl.pallas_call(
        _k1_body,
        grid=(B2, N // BM),
        in_specs=[
            pl.BlockSpec((1, C, BM), lambda b, i: (b + b0, 0, i)),
            pl.BlockSpec((1, C, N), lambda b, i: (b + b0, 0, 0)),
            pl.BlockSpec((1, CP, BM), lambda b, i: (b + b0, 0, i)),
            pl.BlockSpec((1, CP, N), lambda b, i: (b + b0, 0, 0)),
        ],
        out_specs=pl.BlockSpec((1, BM, N), lambda b, i: (b, i, 0)),
        out_shape=jax.ShapeDtypeStruct((B2, N, N), jnp.float32),
        compiler_params=pltpu.CompilerParams(
            dimension_semantics=("parallel", "parallel")),
    )(q_r, k_r, qp_r, kp_r)


def _k2_body(b0, prev_ref, a_hbm, idx_out, s_out,
             idx_vmem, a_buf0, a_buf1, idxbuf, sbuf, sem0, sem1):
    wid = lax.axis_index("c") * 16 + lax.axis_index("s")
    tile_base = wid * RPT
    batch = b0 + wid // TPB
    n0 = (wid % TPB) * RPT

    pltpu.sync_copy(prev_ref.at[pl.ds(batch * N * NK, N * NK)], idx_vmem)

    iota16 = lax.broadcasted_iota(jnp.int32, (16,), 0)

    def chunk_src(ci):
        return a_hbm.at[pl.ds(tile_base + ci * CH, CH)]

    def process(ci, a_chunk):
        n_vec = n0 + ci * CH + iota16
        y = lax.shift_right_logical(n_vec, 5)
        x = lax.bitwise_and(n_vec, 31)
        srcs = [
            n_vec,
            jnp.maximum(y - 1, 0) * 32 + x,
            jnp.minimum(y + 1, 31) * 32 + x,
            y * 32 + jnp.maximum(x - 1, 0),
            y * 32 + jnp.minimum(x + 1, 31),
        ]
        col0 = ci * CH
        for c in range(K5):
            g, t = c // 8, c % 8
            p_val = plsc.load_gather(idx_vmem, [srcs[g] * NK + t])
            py = lax.shift_right_logical(p_val, 5)
            px = lax.bitwise_and(p_val, 31)
            if g == 0:
                oidx = p_val
            elif g == 1:
                oidx = jnp.minimum(py + 1, 31) * 32 + px
            elif g == 2:
                oidx = jnp.maximum(py - 1, 0) * 32 + px
            elif g == 3:
                oidx = py * 32 + jnp.minimum(px + 1, 31)
            else:
                oidx = py * 32 + jnp.maximum(px - 1, 0)
            idxbuf[c, pl.ds(col0, 16)] = oidx
            sbuf[c, pl.ds(col0, 16)] = plsc.load_gather(a_chunk,
                                                        [iota16, oidx])

    pltpu.async_copy(chunk_src(0), a_buf0, sem0)
    pltpu.async_copy(chunk_src(1), a_buf1, sem1)

    def pair_body(i, carry):
        ci0 = 2 * i
        ci1 = 2 * i + 1
        pltpu.make_async_copy(chunk_src(ci0), a_buf0, sem0).wait()
        process(ci0, a_buf0)
        pltpu.async_copy(chunk_src(jnp.minimum(ci0 + 2, NCH - 1)), a_buf0,
                         sem0)
        pltpu.make_async_copy(chunk_src(ci1), a_buf1, sem1).wait()
        process(ci1, a_buf1)
        pltpu.async_copy(chunk_src(jnp.minimum(ci1 + 2, NCH - 1)), a_buf1,
                         sem1)
        return carry

    lax.fori_loop(0, NCH // 2, pair_body, 0)
    pltpu.make_async_copy(chunk_src(NCH - 1), a_buf0, sem0).wait()
    pltpu.make_async_copy(chunk_src(NCH - 1), a_buf1, sem1).wait()

    zero16 = jnp.zeros((16,), jnp.int32)
    zf16 = jnp.zeros((16,), jnp.float32)
    for c in range(K5, W48):
        for j in range(RPT // 16):
            idxbuf[c, pl.ds(j * 16, 16)] = zero16
            sbuf[c, pl.ds(j * 16, 16)] = zf16

    pltpu.sync_copy(idxbuf, idx_out.at[wid])
    pltpu.sync_copy(sbuf, s_out.at[wid])


def _k2(prev_flat, a_half, b0):
    mesh = plsc.VectorSubcoreMesh(core_axis_name="c", subcore_axis_name="s")
    return pl.kernel(
        lambda *refs: _k2_body(b0, *refs),
        out_type=[
            jax.ShapeDtypeStruct((NTILES, W48, RPT), jnp.int32),
            jax.ShapeDtypeStruct((NTILES, W48, RPT), jnp.float32),
        ],
        mesh=mesh,
        compiler_params=pltpu.CompilerParams(needs_layout_passes=False),
        scratch_types=[
            pltpu.VMEM((N * NK,), jnp.int32),
            pltpu.VMEM((CH, N), jnp.float32),
            pltpu.VMEM((CH, N), jnp.float32),
            pltpu.VMEM((W48, RPT), jnp.int32),
            pltpu.VMEM((W48, RPT), jnp.float32),
            pltpu.SemaphoreType.DMA,
            pltpu.SemaphoreType.DMA,
        ],
    )(prev_flat, a_half)


ROWS_ALL = NH * ROWS2


def _k3_body(s_ref, i_ref, w_ref, scat_ref, topk_ref):
    s_enc = lax.bitcast_convert_type(s_ref[...], jnp.int32)
    pmask = (s_enc & 1) == 1
    s = lax.bitcast_convert_type(s_enc & -2, jnp.float32)
    idx = i_ref[...]
    sub = lax.broadcasted_iota(jnp.int32, (T2, W48, RPT), 1)
    tmask = sub < K5

    smooth = jnp.float32(C ** 0.5)
    s_m = jnp.where(tmask, s / smooth, -jnp.inf)
    mx = jnp.max(s_m, axis=1, keepdims=True)
    e = jnp.exp(s_m - mx)
    denom = jnp.sum(e, axis=1, keepdims=True)
    attn = e / denom
    w = jnp.where(tmask & pmask, attn, 0.0)
    conf = jnp.sum(w, axis=1, keepdims=True)

    vals = jnp.where(tmask, attn, -1.0)
    accw = jnp.zeros((T2, W48, RPT), jnp.float32)
    minpos = jnp.full((T2, W48, RPT), W48, jnp.int32)
    rank = jnp.zeros((T2, W48, RPT), jnp.int32)
    for tp in range(K5):
        idx_tp = idx[:, tp:tp + 1, :]
        v_tp = vals[:, tp:tp + 1, :]
        eq = idx == idx_tp
        accw = accw + jnp.where(eq, w[:, tp:tp + 1, :], 0.0)
        minpos = jnp.minimum(minpos, jnp.where(eq, tp, W48))
        beats = (v_tp > vals) | ((v_tp == vals) & (tp < sub))
        rank = rank + beats.astype(jnp.int32)
    first = (minpos == sub) & tmask
    w_comb = jnp.where(first, accw, 0.0)
    scat_ref[...] = jnp.where(first, idx, -1)
    w_ref[...] = jnp.where(sub == (W48 - 1), conf, w_comb)

    rank = jnp.where(tmask, rank, W48)
    rows_k = [jnp.sum(jnp.where(rank == i, idx, 0), axis=1, keepdims=True)
              for i in range(NK)]
    tk = jnp.transpose(jnp.concatenate(rows_k, axis=1), (0, 2, 1))
    topk_ref[...] = tk.reshape(T2 * RPT, NK)


def _k3(s3, idx3, hh, topk_prev=None):
    off = hh * (NTILES // T2)
    body = _k3_body
    in_specs = [
        pl.BlockSpec((T2, W48, RPT), lambda i: (i, 0, 0)),
        pl.BlockSpec((T2, W48, RPT), lambda i: (i, 0, 0)),
    ]
    args = [s3, idx3]
    aliases = {}
    if topk_prev is not None:
        body = lambda s, i, tp, w, sc, tk: _k3_body(s, i, w, sc, tk)
        in_specs.append(pl.BlockSpec(memory_space=pl.ANY))
        args.append(topk_prev)
        aliases = {2: 2}
    return pl.pallas_call(
        body,
        grid=(NTILES // T2,),
        in_specs=in_specs,
        out_specs=[
            pl.BlockSpec((T2, W48, RPT), lambda i: (i, 0, 0)),
            pl.BlockSpec((T2, W48, RPT), lambda i: (i, 0, 0)),
            pl.BlockSpec((T2 * RPT, NK), lambda i: (i + off, 0)),
        ],
        out_shape=[
            jax.ShapeDtypeStruct((NTILES, W48, RPT), jnp.float32),
            jax.ShapeDtypeStruct((NTILES, W48, RPT), jnp.int32),
            jax.ShapeDtypeStruct((ROWS_ALL, NK), jnp.int32),
        ],
        input_output_aliases=aliases,
        compiler_params=pltpu.CompilerParams(
            dimension_semantics=("arbitrary",)),
    )(*args)


def _k4_body(w_hbm, scat_hbm, s_out, wbuf, scatbuf, s_chunk):
    wid = lax.axis_index("c") * 16 + lax.axis_index("s")
    tile_base = wid * RPT

    pltpu.sync_copy(w_hbm.at[wid], wbuf)
    pltpu.sync_copy(scat_hbm.at[wid], scatbuf)

    iota16 = lax.broadcasted_iota(jnp.int32, (16,), 0)
    zf16 = jnp.zeros((16,), jnp.float32)

    def zero_body(j, carry):
        for r in range(CH):
            s_chunk[r, pl.ds(j * 16, 16)] = zf16
        return carry

    lax.fori_loop(0, N // 16, zero_body, 0)

    def chunk_body(ci, carry):
        col0 = ci * CH
        for c in range(K5):
            w_vec = wbuf[c, pl.ds(col0, 16)]
            scat = scatbuf[c, pl.ds(col0, 16)]
            col = jnp.maximum(scat, 0)
            plsc.store_scatter(s_chunk, [iota16, col], w_vec, mask=scat >= 0)
        pltpu.sync_copy(s_chunk,
                        s_out.at[pl.ds(tile_base + ci * CH, CH)])
        for c in range(K5):
            scat = scatbuf[c, pl.ds(col0, 16)]
            col = jnp.maximum(scat, 0)
            plsc.store_scatter(s_chunk, [iota16, col], zf16, mask=scat >= 0)
        return carry

    lax.fori_loop(0, NCH, chunk_body, 0)


def _k4(w3, scat3):
    mesh = plsc.VectorSubcoreMesh(core_axis_name="c", subcore_axis_name="s")
    return pl.kernel(
        _k4_body,
        out_type=jax.ShapeDtypeStruct((ROWS2, N), jnp.float32),
        mesh=mesh,
        compiler_params=pltpu.CompilerParams(needs_layout_passes=False),
        scratch_types=[
            pltpu.VMEM((W48, RPT), jnp.float32),
            pltpu.VMEM((W48, RPT), jnp.int32),
            pltpu.VMEM((CH, N), jnp.float32),
        ],
    )(w3, scat3)


def _k5_body(v_ref, s_ref, o_ref):
    o_ref[0] = lax.dot_general(v_ref[0], s_ref[0], (((1,), (1,)), ((), ())),
                               preferred_element_type=jnp.float32)


def _k5(v_r, s_dense, b0, out_prev=None):
    body = _k5_body
    in_specs = [
        pl.BlockSpec((1, CV, N), lambda b, i: (b + b0, 0, 0)),
        pl.BlockSpec((1, BM, N), lambda b, i: (b, i, 0)),
    ]
    args = [v_r, s_dense]
    aliases = {}
    if out_prev is not None:
        body = lambda v, s, op, o: _k5_body(v, s, o)
        in_specs.append(pl.BlockSpec(memory_space=pl.ANY))
        args.append(out_prev)
        aliases = {2: 0}
    return pl.pallas_call(
        body,
        grid=(B2, N // BM),
        in_specs=in_specs,
        out_specs=pl.BlockSpec((1, CV, BM), lambda b, i: (b + b0, 0, i)),
        out_shape=jax.ShapeDtypeStruct((B, CV, N), jnp.float32),
        input_output_aliases=aliases,
        compiler_params=pltpu.CompilerParams(
            dimension_semantics=("arbitrary", "arbitrary")),
    )(*args)


@jax.jit
def kernel(q, k, q_prune, k_prune, v, prev_attn_top_k_idx):
    q_r = q.reshape(B, C, N)
    k_r = k.reshape(B, C, N)
    qp_r = q_prune.reshape(B, CP, N)
    kp_r = k_prune.reshape(B, CP, N)
    v_r = v.reshape(B, CV, N)
    prev_flat = prev_attn_top_k_idx.reshape(B * N * NK)

    out_full = None
    topk_full = None
    confs = []
    for hh in range(NH):
        b0 = hh * B2
        a_d = _k1(q_r, k_r, qp_r, kp_r, b0)
        idx3, s3 = _k2(prev_flat, a_d.reshape(ROWS2, N), b0)
        w3, scat3, topk_full = _k3(s3, idx3, hh, topk_full)
        s2 = _k4(w3, scat3)
        out_full = _k5(v_r, s2.reshape(B2, N, N), b0, out_full)
        confs.append(w3[:, W48 - 1, :])

    output = out_full.reshape(B, CV, H, W)
    this_attn_top_k_idx = topk_full.reshape(B, N, NK)
    conf = jnp.concatenate(confs, axis=0).reshape(B, 1, H, W)
    return output, this_attn_top_k_idx, conf
